# baseline (speedup 1.0000x reference)
import jax
import jax.numpy as jnp
from jax.experimental import pallas as pl
from jax.experimental.pallas import tpu as pltpu

_F = 26
_D = 100
_K = 16
_GK = _F * _K


def _ffm_block(*refs):
    x_refs = refs[:_F]
    vr_ref, wp_ref, o_ref, e_ref = refs[_F:]
    lin = None
    diag = None
    for f in range(_F):
        xf = x_refs[f][:, 0, 0, :]
        lterm = jnp.sum(xf * wp_ref[f], axis=1)
        lin = lterm if lin is None else lin + lterm
        ef = jnp.dot(xf, vr_ref[f], preferred_element_type=jnp.float32)
        e_ref[f] = ef
        dsl = ef[:, f * _K:(f + 1) * _K]
        dterm = jnp.sum(dsl * dsl, axis=1)
        diag = dterm if diag is None else diag + dterm

    s = None
    for f in range(_F):
        ef = e_ref[f]
        tf = e_ref[:, :, f * _K:(f + 1) * _K]
        tf = jnp.swapaxes(tf, 0, 1).reshape(ef.shape[0], _GK)
        term = jnp.sum(ef * tf, axis=1)
        s = term if s is None else s + term

    o_ref[...] = (lin + 0.5 * (s - diag))[:, None]


def kernel(inputs, w, v):
    b = inputs.shape[0]
    bt = 256
    grid = b // bt

    xr = inputs.reshape(b, _F, 1, _D)
    vr = v.reshape(_F, _F, _D, _K).transpose(1, 2, 0, 3).reshape(_F, _D, _GK)
    wp = w.reshape(_F, 1, _D)

    x_specs = [
        pl.BlockSpec((bt, 1, 1, _D), lambda i, f=f: (i, f, 0, 0))
        for f in range(_F)
    ]
    out = pl.pallas_call(
        _ffm_block,
        grid=(grid,),
        in_specs=x_specs + [
            pl.BlockSpec((_F, _D, _GK), lambda i: (0, 0, 0)),
            pl.BlockSpec((_F, 1, _D), lambda i: (0, 0, 0)),
        ],
        out_specs=pl.BlockSpec((bt, 1), lambda i: (i, 0)),
        out_shape=jax.ShapeDtypeStruct((b, 1), jnp.float32),
        scratch_shapes=[pltpu.VMEM((_F, bt, _GK), jnp.float32)],
    )(*([xr] * _F + [vr, wp]))
    return out


# speedup vs baseline: 1.0927x; 1.0927x over previous
import jax
import jax.numpy as jnp
from jax.experimental import pallas as pl
from jax.experimental.pallas import tpu as pltpu

_F = 26
_D = 100
_K = 16
_GK = _F * _K
_T = _F * _D


def _ffm_block(x_ref, vr_ref, w_ref, o_ref, e_ref):
    x = x_ref[...]
    lin = jnp.sum(x * w_ref[...], axis=1)

    diag = None
    for f in range(_F):
        xf = x[:, f * _D:(f + 1) * _D]
        ef = jnp.dot(xf, vr_ref[f], preferred_element_type=jnp.float32)
        e_ref[f] = ef
        dsl = ef[:, f * _K:(f + 1) * _K]
        dterm = jnp.sum(dsl * dsl, axis=1)
        diag = dterm if diag is None else diag + dterm

    s = None
    for f in range(_F):
        ef = e_ref[f]
        tf = e_ref[:, :, f * _K:(f + 1) * _K]
        tf = jnp.swapaxes(tf, 0, 1).reshape(ef.shape[0], _GK)
        term = jnp.sum(ef * tf, axis=1)
        s = term if s is None else s + term

    o_ref[...] = (lin + 0.5 * (s - diag))[:, None]


def kernel(inputs, w, v):
    b = inputs.shape[0]
    bt = 256
    grid = b // bt

    vr = v.reshape(_F, _F, _D, _K).transpose(1, 2, 0, 3).reshape(_F, _D, _GK)
    wt = w.reshape(1, _T)

    out = pl.pallas_call(
        _ffm_block,
        grid=(grid,),
        in_specs=[
            pl.BlockSpec((bt, _T), lambda i: (i, 0)),
            pl.BlockSpec((_F, _D, _GK), lambda i: (0, 0, 0)),
            pl.BlockSpec((1, _T), lambda i: (0, 0)),
        ],
        out_specs=pl.BlockSpec((bt, 1), lambda i: (i, 0)),
        out_shape=jax.ShapeDtypeStruct((b, 1), jnp.float32),
        scratch_shapes=[pltpu.VMEM((_F, bt, _GK), jnp.float32)],
    )(inputs, vr, wt)
    return out


# speedup vs baseline: 1.1812x; 1.0810x over previous
import jax
import jax.numpy as jnp
from jax.experimental import pallas as pl
from jax.experimental.pallas import tpu as pltpu

_F = 26
_D = 100
_K = 16
_GK = _F * _K
_T = _F * _D


def _ffm_block(x_ref, vr_ref, w_ref, o_ref, e_ref, et_ref):
    x = x_ref[...]
    lin = jnp.sum(x * w_ref[...], axis=1)

    diag = None
    for f in range(_F):
        xf = x[:, f * _D:(f + 1) * _D]
        ef = jnp.dot(xf, vr_ref[f], preferred_element_type=jnp.float32)
        e_ref[f] = ef
        for g in range(_F):
            et_ref[g, :, f * _K:(f + 1) * _K] = ef[:, g * _K:(g + 1) * _K]
        dsl = ef[:, f * _K:(f + 1) * _K]
        dterm = jnp.sum(dsl * dsl, axis=1)
        diag = dterm if diag is None else diag + dterm

    sacc = None
    for f in range(_F):
        prod = e_ref[f] * et_ref[f]
        sacc = prod if sacc is None else sacc + prod
    s = jnp.sum(sacc, axis=1)

    o_ref[...] = (lin + 0.5 * (s - diag))[:, None]


def kernel(inputs, w, v):
    b = inputs.shape[0]
    bt = 256
    grid = b // bt

    vr = v.reshape(_F, _F, _D, _K).transpose(1, 2, 0, 3).reshape(_F, _D, _GK)
    wt = w.reshape(1, _T)

    out = pl.pallas_call(
        _ffm_block,
        grid=(grid,),
        in_specs=[
            pl.BlockSpec((bt, _T), lambda i: (i, 0)),
            pl.BlockSpec((_F, _D, _GK), lambda i: (0, 0, 0)),
            pl.BlockSpec((1, _T), lambda i: (0, 0)),
        ],
        out_specs=pl.BlockSpec((bt, 1), lambda i: (i, 0)),
        out_shape=jax.ShapeDtypeStruct((b, 1), jnp.float32),
        scratch_shapes=[
            pltpu.VMEM((_F, bt, _GK), jnp.float32),
            pltpu.VMEM((_F, bt, _GK), jnp.float32),
        ],
    )(inputs, vr, wt)
    return out


# speedup vs baseline: 5.9043x; 4.9988x over previous
import jax
import jax.numpy as jnp
from jax.experimental import pallas as pl
from jax.experimental.pallas import tpu as pltpu

_F = 26
_D = 100
_DP = 104
_K = 16
_GK = _F * _K
_TP = _F * _DP


def _ffm_block(xt_ref, vrt_ref, wt_ref, o_ref, e_ref):
    xt = xt_ref[...]
    lin = jnp.sum(xt * wt_ref[...], axis=0, keepdims=True)

    for f in range(_F):
        xf = xt[f * _DP:(f + 1) * _DP, :]
        e_ref[f] = jnp.dot(vrt_ref[f], xf,
                           preferred_element_type=jnp.float32)

    acc = None
    for i in range(_F - 1):
        ei = e_ref[i]
        for j in range(i + 1, _F):
            prod = ei[j * _K:(j + 1) * _K, :] * e_ref[j, i * _K:(i + 1) * _K, :]
            acc = prod if acc is None else acc + prod

    o_ref[...] = lin + jnp.sum(acc, axis=0, keepdims=True)


def kernel(inputs, w, v):
    b = inputs.shape[0]
    bt = 256
    grid = b // bt

    xp = jnp.pad(inputs.reshape(b, _F, _D), ((0, 0), (0, 0), (0, _DP - _D)))
    xt = xp.reshape(b, _TP).T
    vrt = v.reshape(_F, _F, _D, _K).transpose(1, 0, 3, 2).reshape(_F, _GK, _D)
    vrt = jnp.pad(vrt, ((0, 0), (0, 0), (0, _DP - _D)))
    wt = jnp.pad(w.reshape(_F, _D), ((0, 0), (0, _DP - _D))).reshape(_TP, 1)

    out = pl.pallas_call(
        _ffm_block,
        grid=(grid,),
        in_specs=[
            pl.BlockSpec((_TP, bt), lambda i: (0, i)),
            pl.BlockSpec((_F, _GK, _DP), lambda i: (0, 0, 0)),
            pl.BlockSpec((_TP, 1), lambda i: (0, 0)),
        ],
        out_specs=pl.BlockSpec((1, bt), lambda i: (0, i)),
        out_shape=jax.ShapeDtypeStruct((1, b), jnp.float32),
        scratch_shapes=[pltpu.VMEM((_F, _GK, bt), jnp.float32)],
    )(xt, vrt, wt)
    return out.reshape(b, 1)


# speedup vs baseline: 5.9473x; 1.0073x over previous
import jax
import jax.numpy as jnp
from jax.experimental import pallas as pl
from jax.experimental.pallas import tpu as pltpu

_F = 26
_D = 100
_DP = 104
_K = 16
_GK = _F * _K
_TP = _F * _DP


def _ffm_block(xt_ref, vrt_ref, wt_ref, o_ref, e_ref):
    xt = xt_ref[...]
    lin = jnp.sum(xt * wt_ref[...], axis=0, keepdims=True)

    for f in range(_F):
        xf = xt[f * _DP:(f + 1) * _DP, :]
        e_ref[f] = jnp.dot(vrt_ref[f], xf,
                           preferred_element_type=jnp.float32)

    acc = None
    for i in range(_F - 1):
        ei = e_ref[i]
        for j in range(i + 1, _F):
            prod = ei[j * _K:(j + 1) * _K, :] * e_ref[j, i * _K:(i + 1) * _K, :]
            acc = prod if acc is None else acc + prod

    o_ref[...] = lin + jnp.sum(acc, axis=0, keepdims=True)


def kernel(inputs, w, v):
    b = inputs.shape[0]
    bt = 512
    grid = b // bt

    xp = jnp.pad(inputs.reshape(b, _F, _D), ((0, 0), (0, 0), (0, _DP - _D)))
    xt = xp.reshape(b, _TP).T
    vrt = v.reshape(_F, _F, _D, _K).transpose(1, 0, 3, 2).reshape(_F, _GK, _D)
    vrt = jnp.pad(vrt, ((0, 0), (0, 0), (0, _DP - _D)))
    wt = jnp.pad(w.reshape(_F, _D), ((0, 0), (0, _DP - _D))).reshape(_TP, 1)

    out = pl.pallas_call(
        _ffm_block,
        grid=(grid,),
        in_specs=[
            pl.BlockSpec((_TP, bt), lambda i: (0, i)),
            pl.BlockSpec((_F, _GK, _DP), lambda i: (0, 0, 0)),
            pl.BlockSpec((_TP, 1), lambda i: (0, 0)),
        ],
        out_specs=pl.BlockSpec((1, bt), lambda i: (0, i)),
        out_shape=jax.ShapeDtypeStruct((1, b), jnp.float32),
        scratch_shapes=[pltpu.VMEM((_F, _GK, bt), jnp.float32)],
    )(xt, vrt, wt)
    return out.reshape(b, 1)


# speedup vs baseline: 6.9339x; 1.1659x over previous
import jax
import jax.numpy as jnp
from jax.experimental import pallas as pl
from jax.experimental.pallas import tpu as pltpu

_F = 26
_D = 100
_DP = 104
_K = 16
_GK = _F * _K
_TP = _F * _DP


def _ffm_block(x_ref, vrt_ref, wt_ref, o_ref, e_ref):
    x = x_ref[...]
    lin = jnp.sum(x * wt_ref[...], axis=1)[None, :]

    for f in range(_F):
        xf = x[:, f * _D:(f + 1) * _D]
        e_ref[f] = jax.lax.dot_general(
            vrt_ref[f], xf, (((1,), (1,)), ((), ())),
            preferred_element_type=jnp.float32)

    acc = None
    for i in range(_F - 1):
        ei = e_ref[i]
        for j in range(i + 1, _F):
            prod = ei[j * _K:(j + 1) * _K, :] * e_ref[j, i * _K:(i + 1) * _K, :]
            acc = prod if acc is None else acc + prod

    o_ref[...] = lin + jnp.sum(acc, axis=0, keepdims=True)


def kernel(inputs, w, v):
    b = inputs.shape[0]
    bt = 512
    grid = b // bt

    vrt = v.reshape(_F, _F, _D, _K).transpose(1, 0, 3, 2).reshape(_F, _GK, _D)
    wt = w.reshape(1, _F * _D)

    out = pl.pallas_call(
        _ffm_block,
        grid=(grid,),
        in_specs=[
            pl.BlockSpec((bt, _F * _D), lambda i: (i, 0)),
            pl.BlockSpec((_F, _GK, _D), lambda i: (0, 0, 0)),
            pl.BlockSpec((1, _F * _D), lambda i: (0, 0)),
        ],
        out_specs=pl.BlockSpec((1, bt), lambda i: (0, i)),
        out_shape=jax.ShapeDtypeStruct((1, b), jnp.float32),
        scratch_shapes=[pltpu.VMEM((_F, _GK, bt), jnp.float32)],
    )(inputs, vrt, wt)
    return out.reshape(b, 1)


# speedup vs baseline: 7.1290x; 1.0281x over previous
import jax
import jax.numpy as jnp
from jax.experimental import pallas as pl
from jax.experimental.pallas import tpu as pltpu

_F = 26
_D = 100
_DP = 104
_K = 16
_GK = _F * _K
_GKA = 424
_TP = _F * _DP


def _ffm_block(x_ref, vrt_ref, o_ref, e_ref):
    x = x_ref[...]

    for f in range(_F):
        xf = x[:, f * _D:(f + 1) * _D]
        e_ref[f] = jax.lax.dot_general(
            vrt_ref[f], xf, (((1,), (1,)), ((), ())),
            preferred_element_type=jnp.float32)

    lin = None
    for f in range(_F):
        lrow = e_ref[f, _GK:_GK + 1, :]
        lin = lrow if lin is None else lin + lrow

    acc = None
    for i in range(_F - 1):
        ei = e_ref[i]
        for j in range(i + 1, _F):
            prod = ei[j * _K:(j + 1) * _K, :] * e_ref[j, i * _K:(i + 1) * _K, :]
            acc = prod if acc is None else acc + prod

    o_ref[...] = lin + jnp.sum(acc, axis=0, keepdims=True)


def kernel(inputs, w, v):
    b = inputs.shape[0]
    bt = 512
    grid = b // bt

    vrt = v.reshape(_F, _F, _D, _K).transpose(1, 0, 3, 2).reshape(_F, _GK, _D)
    vrt = jnp.concatenate(
        [vrt, w.reshape(_F, 1, _D),
         jnp.zeros((_F, _GKA - _GK - 1, _D), vrt.dtype)], axis=1)

    out = pl.pallas_call(
        _ffm_block,
        grid=(grid,),
        in_specs=[
            pl.BlockSpec((bt, _F * _D), lambda i: (i, 0)),
            pl.BlockSpec((_F, _GKA, _D), lambda i: (0, 0, 0)),
        ],
        out_specs=pl.BlockSpec((1, bt), lambda i: (0, i)),
        out_shape=jax.ShapeDtypeStruct((1, b), jnp.float32),
        scratch_shapes=[pltpu.VMEM((_F, _GKA, bt), jnp.float32)],
    )(inputs, vrt)
    return out.reshape(b, 1)
